# baseline (speedup 1.0000x reference)
import functools

import jax
import jax.numpy as jnp
from jax import lax
from jax.experimental import pallas as pl
from jax.experimental.pallas import tpu as pltpu
from jax.experimental.pallas import tpu_sc as plsc

_VOCAB = 100000
_EMB = 16
_CTX = 20
_BATCH = 1024
_NIDX = _CTX * _BATCH

_NC = 2
_NS = 16
_NW = _NC * _NS
_B_PER_W = _NIDX // _NW
_CHUNK = 128
_NCHUNK = _B_PER_W // _CHUNK
_DPAD = 128

_TV = 2048
_NT = (_VOCAB + _TV - 1) // _TV
_BB = 512
_NB = _BATCH // _BB
_B8 = _BB // 8


def _sc_gather(em, idx3):
    mesh = plsc.VectorSubcoreMesh(core_axis_name="c", subcore_axis_name="s")

    @functools.partial(
        pl.kernel,
        mesh=mesh,
        out_type=jax.ShapeDtypeStruct((_NIDX, _DPAD), jnp.float32),
        scratch_types=[
            pltpu.VMEM((_NCHUNK, _CHUNK), jnp.int32),
            pltpu.VMEM((_CHUNK, _DPAD), jnp.float32),
            pltpu.SemaphoreType.DMA,
        ],
    )
    def k(table_hbm, idx_hbm, out_hbm, idx_v, rows_v, sem):
        wid = lax.axis_index("s") * _NC + lax.axis_index("c")
        base = wid * _B_PER_W
        pltpu.sync_copy(idx_hbm.at[wid], idx_v)

        @pl.loop(0, _NCHUNK)
        def _(j):
            pltpu.async_copy(table_hbm.at[idx_v.at[j]], rows_v, sem).wait()
            pltpu.sync_copy(rows_v, out_hbm.at[pl.ds(base + j * _CHUNK, _CHUNK)])

    return k(em, idx3)


def _reduce_kernel(x4, w1r, w2):
    def body(x_ref, w1_ref, w2_ref, h_ref, s_ref, hs_ref):
        t = pl.program_id(1)

        @pl.when(t == 0)
        def _():
            x4v = x_ref[...]
            acc = jnp.zeros((_BB, _EMB), jnp.float32)
            for j in range(_CTX):
                xj = x4v[:, j].reshape(_BB, _DPAD).astype(jnp.bfloat16)
                acc = acc + jnp.dot(xj, w1_ref[j],
                                    preferred_element_type=jnp.float32)
            hb = jnp.maximum(acc, 0.0).astype(jnp.bfloat16)
            hs_ref[...] = hb
            h_ref[...] = hb
            s_ref[...] = jnp.zeros_like(s_ref)

        wt = jnp.transpose(w2_ref[...]).astype(jnp.bfloat16)
        logits = jnp.dot(hs_ref[...], wt,
                         preferred_element_type=jnp.float32)

        @pl.when(t < _NT - 1)
        def _():
            s_ref[...] = s_ref[...] + jnp.sum(jnp.exp(logits), axis=1,
                                              keepdims=True)

        @pl.when(t == _NT - 1)
        def _():
            col = lax.broadcasted_iota(jnp.int32, (_BB, _TV), 1)
            e = jnp.where(col < _VOCAB - (_NT - 1) * _TV, jnp.exp(logits), 0.0)
            s_ref[...] = s_ref[...] + jnp.sum(e, axis=1, keepdims=True)

    return pl.pallas_call(
        body,
        grid=(_NB, _NT),
        in_specs=[
            pl.BlockSpec((_B8, _CTX, 8, _DPAD), lambda b, t: (b, 0, 0, 0)),
            pl.BlockSpec((_CTX, _DPAD, _EMB), lambda b, t: (0, 0, 0)),
            pl.BlockSpec((_TV, _EMB), lambda b, t: (t, 0)),
        ],
        out_specs=[
            pl.BlockSpec((_BB, _EMB), lambda b, t: (b, 0)),
            pl.BlockSpec((_BB, 1), lambda b, t: (b, 0)),
        ],
        out_shape=[
            jax.ShapeDtypeStruct((_BATCH, _EMB), jnp.bfloat16),
            jax.ShapeDtypeStruct((_BATCH, 1), jnp.float32),
        ],
        scratch_shapes=[pltpu.VMEM((_BB, _EMB), jnp.bfloat16)],
        compiler_params=pltpu.CompilerParams(
            dimension_semantics=("parallel", "arbitrary"),
        ),
    )(x4, w1r, w2)


def _write_kernel(h, s, w2):
    def body(h_ref, s_ref, w2_ref, o_ref):
        wt = jnp.transpose(w2_ref[...]).astype(jnp.bfloat16)
        logits = jnp.dot(h_ref[...], wt,
                         preferred_element_type=jnp.float32)
        o_ref[...] = logits - jnp.log(s_ref[...])

    return pl.pallas_call(
        body,
        grid=(_NB, _NT),
        in_specs=[
            pl.BlockSpec((_BB, _EMB), lambda b, t: (b, 0)),
            pl.BlockSpec((_BB, 1), lambda b, t: (b, 0)),
            pl.BlockSpec((_TV, _EMB), lambda b, t: (t, 0)),
        ],
        out_specs=pl.BlockSpec((_BB, _TV), lambda b, t: (b, t)),
        out_shape=jax.ShapeDtypeStruct((_BATCH, _VOCAB), jnp.float32),
        compiler_params=pltpu.CompilerParams(
            dimension_semantics=("parallel", "arbitrary"),
        ),
    )(h, s, w2)


def kernel(inputs, em, W1, W2):
    idxp = jnp.transpose(inputs.reshape(_BATCH // 8, 8, _CTX), (0, 2, 1))
    idx3 = idxp.reshape(_NW, _NCHUNK, _CHUNK)
    em_pad = jnp.pad(em, ((0, 0), (0, _DPAD - _EMB)))
    rows = _sc_gather(em_pad, idx3)
    x4 = rows.reshape(_BATCH // 8, _CTX, 8, _DPAD)
    w1r = jnp.pad(W1.T.reshape(_CTX, _EMB, _EMB),
                  ((0, 0), (0, _DPAD - _EMB), (0, 0))).astype(jnp.bfloat16)
    h, s = _reduce_kernel(x4, w1r, W2)
    return _write_kernel(h, s, W2)


# speedup vs baseline: 2.0886x; 2.0886x over previous
import functools

import jax
import jax.numpy as jnp
from jax import lax
from jax.experimental import pallas as pl
from jax.experimental.pallas import tpu as pltpu
from jax.experimental.pallas import tpu_sc as plsc

_VOCAB = 100000
_EMB = 16
_CTX = 20
_BATCH = 1024
_NIDX = _CTX * _BATCH

_NC = 2
_NS = 16
_NW = _NC * _NS
_B_PER_W = _NIDX // _NW
_CHUNK = 128
_NCHUNK = _B_PER_W // _CHUNK
_DPAD = 128

_TV = 2048
_NT = (_VOCAB + _TV - 1) // _TV
_BB = 512
_NB = _BATCH // _BB
_B8 = _BB // 8


def _sc_gather(em, idx3):
    mesh = plsc.VectorSubcoreMesh(core_axis_name="c", subcore_axis_name="s")

    @functools.partial(
        pl.kernel,
        mesh=mesh,
        out_type=jax.ShapeDtypeStruct((_NIDX, _DPAD), jnp.float32),
        scratch_types=[
            pltpu.VMEM((_NCHUNK, _CHUNK), jnp.int32),
            pltpu.VMEM((_CHUNK, _DPAD), jnp.float32),
            pltpu.SemaphoreType.DMA,
        ],
    )
    def k(table_hbm, idx_hbm, out_hbm, idx_v, rows_v, sem):
        wid = lax.axis_index("s") * _NC + lax.axis_index("c")
        base = wid * _B_PER_W
        pltpu.sync_copy(idx_hbm.at[wid], idx_v)

        @pl.loop(0, _NCHUNK)
        def _(j):
            pltpu.async_copy(table_hbm.at[idx_v.at[j]], rows_v, sem).wait()
            pltpu.sync_copy(rows_v, out_hbm.at[pl.ds(base + j * _CHUNK, _CHUNK)])

    return k(em, idx3)


def _reduce_kernel(x4, w1r, w2t):
    def body(x_ref, w1_ref, w2_ref, h_ref, s_ref, hs_ref):
        t = pl.program_id(1)

        @pl.when(t == 0)
        def _():
            x4v = x_ref[...]
            acc = jnp.zeros((_BB, _EMB), jnp.float32)
            for j in range(_CTX):
                xj = x4v[:, j].reshape(_BB, _DPAD).astype(jnp.bfloat16)
                acc = acc + jnp.dot(xj, w1_ref[j],
                                    preferred_element_type=jnp.float32)
            hb = jnp.transpose(jnp.maximum(acc, 0.0)).astype(jnp.bfloat16)
            hs_ref[...] = hb
            h_ref[...] = hb
            s_ref[...] = jnp.zeros_like(s_ref)

        logits = lax.dot_general(w2_ref[...], hs_ref[...],
                                 (((0,), (0,)), ((), ())),
                                 preferred_element_type=jnp.float32)

        @pl.when(t < _NT - 1)
        def _():
            e = jnp.sum(jnp.exp(logits), axis=0, keepdims=True)
            s_ref[...] = s_ref[...] + jnp.broadcast_to(e, (8, _BB))

        @pl.when(t == _NT - 1)
        def _():
            row = lax.broadcasted_iota(jnp.int32, (_TV, _BB), 0)
            e = jnp.where(row < _VOCAB - (_NT - 1) * _TV, jnp.exp(logits), 0.0)
            e = jnp.sum(e, axis=0, keepdims=True)
            s_ref[...] = s_ref[...] + jnp.broadcast_to(e, (8, _BB))

    return pl.pallas_call(
        body,
        grid=(_NB, _NT),
        in_specs=[
            pl.BlockSpec((_B8, _CTX, 8, _DPAD), lambda b, t: (b, 0, 0, 0)),
            pl.BlockSpec((_CTX, _DPAD, _EMB), lambda b, t: (0, 0, 0)),
            pl.BlockSpec((_EMB, _TV), lambda b, t: (0, t)),
        ],
        out_specs=[
            pl.BlockSpec((_EMB, _BB), lambda b, t: (0, b)),
            pl.BlockSpec((8, _BB), lambda b, t: (0, b)),
        ],
        out_shape=[
            jax.ShapeDtypeStruct((_EMB, _BATCH), jnp.bfloat16),
            jax.ShapeDtypeStruct((8, _BATCH), jnp.float32),
        ],
        scratch_shapes=[pltpu.VMEM((_EMB, _BB), jnp.bfloat16)],
        compiler_params=pltpu.CompilerParams(
            dimension_semantics=("parallel", "arbitrary"),
        ),
    )(x4, w1r, w2t)


def _write_kernel(ht, s, w2t):
    def body(h_ref, s_ref, w2_ref, o_ref):
        logits = lax.dot_general(w2_ref[...], h_ref[...],
                                 (((0,), (0,)), ((), ())),
                                 preferred_element_type=jnp.float32)
        o_ref[...] = logits - jnp.log(s_ref[0:1, :])

    return pl.pallas_call(
        body,
        grid=(_NB, _NT),
        in_specs=[
            pl.BlockSpec((_EMB, _BB), lambda b, t: (0, b)),
            pl.BlockSpec((8, _BB), lambda b, t: (0, b)),
            pl.BlockSpec((_EMB, _TV), lambda b, t: (0, t)),
        ],
        out_specs=pl.BlockSpec((_TV, _BB), lambda b, t: (t, b)),
        out_shape=jax.ShapeDtypeStruct((_VOCAB, _BATCH), jnp.float32),
        compiler_params=pltpu.CompilerParams(
            dimension_semantics=("parallel", "arbitrary"),
        ),
    )(ht, s, w2t)


def kernel(inputs, em, W1, W2):
    idxp = jnp.transpose(inputs.reshape(_BATCH // 8, 8, _CTX), (0, 2, 1))
    idx3 = idxp.reshape(_NW, _NCHUNK, _CHUNK)
    em_pad = jnp.pad(em, ((0, 0), (0, _DPAD - _EMB)))
    rows = _sc_gather(em_pad, idx3)
    x4 = rows.reshape(_BATCH // 8, _CTX, 8, _DPAD)
    w1r = jnp.pad(W1.T.reshape(_CTX, _EMB, _EMB),
                  ((0, 0), (0, _DPAD - _EMB), (0, 0))).astype(jnp.bfloat16)
    w2t = W2.T.astype(jnp.bfloat16)
    ht, s = _reduce_kernel(x4, w1r, w2t)
    outt = _write_kernel(ht, s, w2t)
    return outt.T


# speedup vs baseline: 2.3316x; 1.1164x over previous
import functools

import jax
import jax.numpy as jnp
from jax import lax
from jax.experimental import pallas as pl
from jax.experimental.pallas import tpu as pltpu
from jax.experimental.pallas import tpu_sc as plsc

_VOCAB = 100000
_EMB = 16
_CTX = 20
_BATCH = 1024
_NIDX = _CTX * _BATCH

_NC = 2
_NS = 16
_NW = _NC * _NS
_B_PER_W = _NIDX // _NW
_CHUNK = 128
_NCHUNK = _B_PER_W // _CHUNK
_DPAD = 128

_TV = 2048
_NT = (_VOCAB + _TV - 1) // _TV
_BB = 512
_NB = _BATCH // _BB
_B8 = _BB // 8


def _sc_gather(em, idx3):
    mesh = plsc.VectorSubcoreMesh(core_axis_name="c", subcore_axis_name="s")

    @functools.partial(
        pl.kernel,
        mesh=mesh,
        out_type=jax.ShapeDtypeStruct((_NIDX, _DPAD), jnp.float32),
        scratch_types=[
            pltpu.VMEM((_NCHUNK, _CHUNK), jnp.int32),
            pltpu.VMEM((_CHUNK, _DPAD), jnp.float32),
            pltpu.SemaphoreType.DMA,
        ],
    )
    def k(table_hbm, idx_hbm, out_hbm, idx_v, rows_v, sem):
        wid = lax.axis_index("s") * _NC + lax.axis_index("c")
        base = wid * _B_PER_W
        pltpu.sync_copy(idx_hbm.at[wid], idx_v)

        @pl.loop(0, _NCHUNK)
        def _(j):
            pltpu.async_copy(table_hbm.at[idx_v.at[j]], rows_v, sem).wait()
            pltpu.sync_copy(rows_v, out_hbm.at[pl.ds(base + j * _CHUNK, _CHUNK)])

    return k(em, idx3)


def _reduce_kernel(x4, w1r, w2t):
    def body(x_ref, w1_ref, w2_ref, h_ref, s_ref, hs_ref):
        t = pl.program_id(1)

        @pl.when(t == 0)
        def _():
            x4v = x_ref[...]
            acc = jnp.zeros((_BB, _EMB), jnp.float32)
            for j in range(_CTX):
                xj = x4v[:, j].reshape(_BB, _DPAD).astype(jnp.bfloat16)
                acc = acc + jnp.dot(xj, w1_ref[j],
                                    preferred_element_type=jnp.float32)
            hb = jnp.transpose(jnp.maximum(acc, 0.0) * (1.0 / 16.0))
            hb = hb.astype(jnp.float8_e4m3fn)
            hs_ref[...] = hb
            h_ref[...] = hb
            s_ref[...] = jnp.zeros_like(s_ref)

        logits = lax.dot_general(w2_ref[...], hs_ref[...],
                                 (((0,), (0,)), ((), ())),
                                 preferred_element_type=jnp.float32)

        @pl.when(t < _NT - 1)
        def _():
            e = jnp.sum(jnp.exp(logits), axis=0, keepdims=True)
            s_ref[...] = s_ref[...] + jnp.broadcast_to(e, (8, _BB))

        @pl.when(t == _NT - 1)
        def _():
            row = lax.broadcasted_iota(jnp.int32, (_TV, _BB), 0)
            e = jnp.where(row < _VOCAB - (_NT - 1) * _TV, jnp.exp(logits), 0.0)
            e = jnp.sum(e, axis=0, keepdims=True)
            s_ref[...] = s_ref[...] + jnp.broadcast_to(e, (8, _BB))

    return pl.pallas_call(
        body,
        grid=(_NB, _NT),
        in_specs=[
            pl.BlockSpec((_B8, _CTX, 8, _DPAD), lambda b, t: (b, 0, 0, 0)),
            pl.BlockSpec((_CTX, _DPAD, _EMB), lambda b, t: (0, 0, 0)),
            pl.BlockSpec((_EMB, _TV), lambda b, t: (0, t)),
        ],
        out_specs=[
            pl.BlockSpec((_EMB, _BB), lambda b, t: (0, b)),
            pl.BlockSpec((8, _BB), lambda b, t: (0, b)),
        ],
        out_shape=[
            jax.ShapeDtypeStruct((_EMB, _BATCH), jnp.float8_e4m3fn),
            jax.ShapeDtypeStruct((8, _BATCH), jnp.float32),
        ],
        scratch_shapes=[pltpu.VMEM((_EMB, _BB), jnp.float8_e4m3fn)],
        compiler_params=pltpu.CompilerParams(
            dimension_semantics=("parallel", "arbitrary"),
        ),
    )(x4, w1r, w2t)


def _write_kernel(ht, s, w2t):
    def body(h_ref, s_ref, w2_ref, o_ref):
        logits = lax.dot_general(w2_ref[...], h_ref[...],
                                 (((0,), (0,)), ((), ())),
                                 preferred_element_type=jnp.float32)
        o_ref[...] = logits - jnp.log(s_ref[0:1, :])

    return pl.pallas_call(
        body,
        grid=(_NB, _NT),
        in_specs=[
            pl.BlockSpec((_EMB, _BB), lambda b, t: (0, b)),
            pl.BlockSpec((8, _BB), lambda b, t: (0, b)),
            pl.BlockSpec((_EMB, _TV), lambda b, t: (0, t)),
        ],
        out_specs=pl.BlockSpec((_TV, _BB), lambda b, t: (t, b)),
        out_shape=jax.ShapeDtypeStruct((_VOCAB, _BATCH), jnp.float32),
        compiler_params=pltpu.CompilerParams(
            dimension_semantics=("parallel", "arbitrary"),
        ),
    )(ht, s, w2t)


def kernel(inputs, em, W1, W2):
    idxp = jnp.transpose(inputs.reshape(_BATCH // 8, 8, _CTX), (0, 2, 1))
    idx3 = idxp.reshape(_NW, _NCHUNK, _CHUNK)
    em_pad = jnp.pad(em, ((0, 0), (0, _DPAD - _EMB)))
    rows = _sc_gather(em_pad, idx3)
    x4 = rows.reshape(_BATCH // 8, _CTX, 8, _DPAD)
    w1r = jnp.pad(W1.T.reshape(_CTX, _EMB, _EMB),
                  ((0, 0), (0, _DPAD - _EMB), (0, 0))).astype(jnp.bfloat16)
    w2t = (W2.T * 16.0).astype(jnp.float8_e4m3fn)
    ht, s = _reduce_kernel(x4, w1r, w2t)
    outt = _write_kernel(ht, s, w2t)
    return outt.T


# speedup vs baseline: 2.6523x; 1.1375x over previous
import functools

import jax
import jax.numpy as jnp
from jax import lax
from jax.experimental import pallas as pl
from jax.experimental.pallas import tpu as pltpu
from jax.experimental.pallas import tpu_sc as plsc

_VOCAB = 100000
_EMB = 16
_CTX = 20
_BATCH = 1024
_NIDX = _CTX * _BATCH

_NC = 2
_NS = 16
_NW = _NC * _NS
_B_PER_W = _NIDX // _NW
_CHUNK = 128
_NCHUNK = _B_PER_W // _CHUNK
_DPAD = 128

_TV = 2048
_NT = (_VOCAB + _TV - 1) // _TV
_BB = 512
_NB = _BATCH // _BB
_B8 = _BB // 8


def _sc_gather(em, idx3):
    mesh = plsc.VectorSubcoreMesh(core_axis_name="c", subcore_axis_name="s")

    @functools.partial(
        pl.kernel,
        mesh=mesh,
        out_type=jax.ShapeDtypeStruct((_NIDX, _DPAD), jnp.float32),
        scratch_types=[
            pltpu.VMEM((_NCHUNK, _CHUNK), jnp.int32),
            pltpu.VMEM((_CHUNK, _DPAD), jnp.float32),
            pltpu.SemaphoreType.DMA,
        ],
    )
    def k(table_hbm, idx_hbm, out_hbm, idx_v, rows_v, sem):
        wid = lax.axis_index("s") * _NC + lax.axis_index("c")
        base = wid * _B_PER_W
        pltpu.sync_copy(idx_hbm.at[wid], idx_v)

        @pl.loop(0, _NCHUNK)
        def _(j):
            pltpu.async_copy(table_hbm.at[idx_v.at[j]], rows_v, sem).wait()
            pltpu.sync_copy(rows_v, out_hbm.at[pl.ds(base + j * _CHUNK, _CHUNK)])

    return k(em, idx3)


def _reduce_kernel(x4, w1r, w2t):
    def body(x_ref, w1_ref, w2_ref, h_ref, s_ref, hs_ref):
        t = pl.program_id(1)

        @pl.when(t == 0)
        def _():
            x4v = x_ref[...]
            acc = jnp.zeros((_BB, _EMB), jnp.float32)
            for j in range(_CTX):
                xj = x4v[:, j].reshape(_BB, _DPAD).astype(jnp.bfloat16)
                acc = acc + jnp.dot(xj, w1_ref[j],
                                    preferred_element_type=jnp.float32)
            hb = jnp.transpose(jnp.maximum(acc, 0.0) * (1.0 / 16.0))
            hb = hb.astype(jnp.float8_e4m3fn)
            hs_ref[...] = hb
            h_ref[...] = hb
            s_ref[...] = jnp.full_like(s_ref, float(_VOCAB - _NT * _TV))

        logits = lax.dot_general(w2_ref[...], hs_ref[...],
                                 (((0,), (0,)), ((), ())),
                                 preferred_element_type=jnp.float32)
        e = jnp.exp(logits.astype(jnp.bfloat16)).astype(jnp.float32)
        e = jnp.sum(e, axis=0, keepdims=True)
        s_ref[...] = s_ref[...] + jnp.broadcast_to(e, (8, _BB))

    return pl.pallas_call(
        body,
        grid=(_NB, _NT),
        in_specs=[
            pl.BlockSpec((_B8, _CTX, 8, _DPAD), lambda b, t: (b, 0, 0, 0)),
            pl.BlockSpec((_CTX, _DPAD, _EMB), lambda b, t: (0, 0, 0)),
            pl.BlockSpec((_EMB, _TV), lambda b, t: (0, t)),
        ],
        out_specs=[
            pl.BlockSpec((_EMB, _BB), lambda b, t: (0, b)),
            pl.BlockSpec((8, _BB), lambda b, t: (0, b)),
        ],
        out_shape=[
            jax.ShapeDtypeStruct((_EMB, _BATCH), jnp.float8_e4m3fn),
            jax.ShapeDtypeStruct((8, _BATCH), jnp.float32),
        ],
        scratch_shapes=[pltpu.VMEM((_EMB, _BB), jnp.float8_e4m3fn)],
        compiler_params=pltpu.CompilerParams(
            dimension_semantics=("parallel", "arbitrary"),
        ),
    )(x4, w1r, w2t)


def _write_kernel(ht, s, w2t):
    def body(h_ref, s_ref, w2_ref, o_ref):
        logits = lax.dot_general(w2_ref[...], h_ref[...],
                                 (((0,), (0,)), ((), ())),
                                 preferred_element_type=jnp.float32)
        o_ref[...] = logits - jnp.log(s_ref[0:1, :])

    return pl.pallas_call(
        body,
        grid=(_NB, _NT),
        in_specs=[
            pl.BlockSpec((_EMB, _BB), lambda b, t: (0, b)),
            pl.BlockSpec((8, _BB), lambda b, t: (0, b)),
            pl.BlockSpec((_EMB, _TV), lambda b, t: (0, t)),
        ],
        out_specs=pl.BlockSpec((_TV, _BB), lambda b, t: (t, b)),
        out_shape=jax.ShapeDtypeStruct((_VOCAB, _BATCH), jnp.float32),
        compiler_params=pltpu.CompilerParams(
            dimension_semantics=("parallel", "arbitrary"),
        ),
    )(ht, s, w2t)


def kernel(inputs, em, W1, W2):
    idxp = jnp.transpose(inputs.reshape(_BATCH // 8, 8, _CTX), (0, 2, 1))
    idx3 = idxp.reshape(_NW, _NCHUNK, _CHUNK)
    em_pad = jnp.pad(em, ((0, 0), (0, _DPAD - _EMB)))
    rows = _sc_gather(em_pad, idx3)
    x4 = rows.reshape(_BATCH // 8, _CTX, 8, _DPAD)
    w1r = jnp.pad(W1.T.reshape(_CTX, _EMB, _EMB),
                  ((0, 0), (0, _DPAD - _EMB), (0, 0))).astype(jnp.bfloat16)
    w2t = (W2.T * 16.0).astype(jnp.float8_e4m3fn)
    w2t = jnp.pad(w2t, ((0, 0), (0, _NT * _TV - _VOCAB)))
    ht, s = _reduce_kernel(x4, w1r, w2t)
    outt = _write_kernel(ht, s, w2t)
    return outt.T


# speedup vs baseline: 2.8046x; 1.0574x over previous
import functools

import jax
import jax.numpy as jnp
from jax import lax
from jax.experimental import pallas as pl
from jax.experimental.pallas import tpu as pltpu
from jax.experimental.pallas import tpu_sc as plsc

_VOCAB = 100000
_EMB = 16
_CTX = 20
_BATCH = 1024
_NIDX = _CTX * _BATCH

_NC = 2
_NS = 16
_NW = _NC * _NS
_B_PER_W = _NIDX // _NW
_CHUNK = 128
_NCHUNK = _B_PER_W // _CHUNK
_DPAD = 128

_TV = 4096
_NT = (_VOCAB + _TV - 1) // _TV
_BB = 512
_NB = _BATCH // _BB
_B8 = _BB // 8


def _sc_gather(em, idx3):
    mesh = plsc.VectorSubcoreMesh(core_axis_name="c", subcore_axis_name="s")

    @functools.partial(
        pl.kernel,
        mesh=mesh,
        out_type=jax.ShapeDtypeStruct((_NIDX, _DPAD), jnp.float32),
        scratch_types=[
            pltpu.VMEM((_NCHUNK, _CHUNK), jnp.int32),
            pltpu.VMEM((_CHUNK, _DPAD), jnp.float32),
            pltpu.SemaphoreType.DMA,
        ],
    )
    def k(table_hbm, idx_hbm, out_hbm, idx_v, rows_v, sem):
        wid = lax.axis_index("s") * _NC + lax.axis_index("c")
        base = wid * _B_PER_W
        pltpu.sync_copy(idx_hbm.at[wid], idx_v)

        @pl.loop(0, _NCHUNK)
        def _(j):
            pltpu.async_copy(table_hbm.at[idx_v.at[j]], rows_v, sem).wait()
            pltpu.sync_copy(rows_v, out_hbm.at[pl.ds(base + j * _CHUNK, _CHUNK)])

    return k(em, idx3)


def _reduce_kernel(x4, w1r, w2t):
    def body(x_ref, w1_ref, w2_ref, h_ref, s_ref, hs_ref):
        t = pl.program_id(1)

        @pl.when(t == 0)
        def _():
            x4v = x_ref[...]
            acc = jnp.zeros((_BB, _EMB), jnp.float32)
            for j in range(_CTX):
                xj = x4v[:, j].reshape(_BB, _DPAD).astype(jnp.bfloat16)
                acc = acc + jnp.dot(xj, w1_ref[j],
                                    preferred_element_type=jnp.float32)
            hb = jnp.transpose(jnp.maximum(acc, 0.0) * (1.0 / 16.0))
            hb = hb.astype(jnp.float8_e4m3fn)
            hs_ref[...] = hb
            h_ref[...] = hb
            s_ref[...] = jnp.full_like(s_ref, float(_VOCAB - _NT * _TV))

        logits = lax.dot_general(w2_ref[...], hs_ref[...],
                                 (((0,), (0,)), ((), ())),
                                 preferred_element_type=jnp.float32)
        e = jnp.exp(logits.astype(jnp.bfloat16)).astype(jnp.float32)
        e = jnp.sum(e, axis=0, keepdims=True)
        s_ref[...] = s_ref[...] + jnp.broadcast_to(e, (8, _BB))

    return pl.pallas_call(
        body,
        grid=(_NB, _NT),
        in_specs=[
            pl.BlockSpec((_B8, _CTX, 8, _DPAD), lambda b, t: (b, 0, 0, 0)),
            pl.BlockSpec((_CTX, _DPAD, _EMB), lambda b, t: (0, 0, 0)),
            pl.BlockSpec((_EMB, _TV), lambda b, t: (0, t)),
        ],
        out_specs=[
            pl.BlockSpec((_EMB, _BB), lambda b, t: (0, b)),
            pl.BlockSpec((8, _BB), lambda b, t: (0, b)),
        ],
        out_shape=[
            jax.ShapeDtypeStruct((_EMB, _BATCH), jnp.float8_e4m3fn),
            jax.ShapeDtypeStruct((8, _BATCH), jnp.float32),
        ],
        scratch_shapes=[pltpu.VMEM((_EMB, _BB), jnp.float8_e4m3fn)],
        compiler_params=pltpu.CompilerParams(
            dimension_semantics=("parallel", "arbitrary"),
        ),
    )(x4, w1r, w2t)


def _write_kernel(ht, s, w2t):
    def body(h_ref, s_ref, w2_ref, o_ref):
        logits = lax.dot_general(w2_ref[...], h_ref[...],
                                 (((0,), (0,)), ((), ())),
                                 preferred_element_type=jnp.float32)
        o_ref[...] = logits - jnp.log(s_ref[0:1, :])

    return pl.pallas_call(
        body,
        grid=(_NB, _NT),
        in_specs=[
            pl.BlockSpec((_EMB, _BB), lambda b, t: (0, b)),
            pl.BlockSpec((8, _BB), lambda b, t: (0, b)),
            pl.BlockSpec((_EMB, _TV), lambda b, t: (0, t)),
        ],
        out_specs=pl.BlockSpec((_TV, _BB), lambda b, t: (t, b)),
        out_shape=jax.ShapeDtypeStruct((_VOCAB, _BATCH), jnp.float32),
        compiler_params=pltpu.CompilerParams(
            dimension_semantics=("parallel", "arbitrary"),
        ),
    )(ht, s, w2t)


def kernel(inputs, em, W1, W2):
    idxp = jnp.transpose(inputs.reshape(_BATCH // 8, 8, _CTX), (0, 2, 1))
    idx3 = idxp.reshape(_NW, _NCHUNK, _CHUNK)
    em_pad = jnp.pad(em, ((0, 0), (0, _DPAD - _EMB)))
    rows = _sc_gather(em_pad, idx3)
    x4 = rows.reshape(_BATCH // 8, _CTX, 8, _DPAD)
    w1r = jnp.pad(W1.T.reshape(_CTX, _EMB, _EMB),
                  ((0, 0), (0, _DPAD - _EMB), (0, 0))).astype(jnp.bfloat16)
    w2t = (W2.T * 16.0).astype(jnp.float8_e4m3fn)
    w2t = jnp.pad(w2t, ((0, 0), (0, _NT * _TV - _VOCAB)))
    ht, s = _reduce_kernel(x4, w1r, w2t)
    outt = _write_kernel(ht, s, w2t)
    return outt.T


# speedup vs baseline: 2.8092x; 1.0017x over previous
import functools

import jax
import jax.numpy as jnp
from jax import lax
from jax.experimental import pallas as pl
from jax.experimental.pallas import tpu as pltpu
from jax.experimental.pallas import tpu_sc as plsc

_VOCAB = 100000
_EMB = 16
_CTX = 20
_BATCH = 1024
_NIDX = _CTX * _BATCH

_NC = 2
_NS = 16
_NW = _NC * _NS
_B_PER_W = _NIDX // _NW
_CHUNK = 128
_NCHUNK = _B_PER_W // _CHUNK
_DPAD = 128

_TV = 4096
_NT = (_VOCAB + _TV - 1) // _TV
_BB = 512
_NB = _BATCH // _BB
_B8 = _BB // 8


def _sc_gather(em, idx3):
    mesh = plsc.VectorSubcoreMesh(core_axis_name="c", subcore_axis_name="s")

    @functools.partial(
        pl.kernel,
        mesh=mesh,
        out_type=jax.ShapeDtypeStruct((_NIDX, _DPAD), jnp.float32),
        scratch_types=[
            pltpu.VMEM((_NCHUNK, _CHUNK), jnp.int32),
            pltpu.VMEM((_CHUNK, _DPAD), jnp.float32),
            pltpu.SemaphoreType.DMA,
        ],
    )
    def k(table_hbm, idx_hbm, out_hbm, idx_v, rows_v, sem):
        wid = lax.axis_index("s") * _NC + lax.axis_index("c")
        base = wid * _B_PER_W
        pltpu.sync_copy(idx_hbm.at[wid], idx_v)

        @pl.loop(0, _NCHUNK)
        def _(j):
            pltpu.async_copy(table_hbm.at[idx_v.at[j]], rows_v, sem).wait()
            pltpu.sync_copy(rows_v, out_hbm.at[pl.ds(base + j * _CHUNK, _CHUNK)])

    return k(em, idx3)


_TVC = 4096
_NTC = (_VOCAB + _TVC - 1) // _TVC


def _pad_table(emt):
    def body(e_ref, o_ref):
        o_ref[:, :_EMB] = jnp.transpose(e_ref[...])
        o_ref[:, _EMB:] = jnp.zeros((_TVC, _DPAD - _EMB), jnp.float32)

    return pl.pallas_call(
        body,
        grid=(_NTC,),
        in_specs=[pl.BlockSpec((_EMB, _TVC), lambda i: (0, i))],
        out_specs=pl.BlockSpec((_TVC, _DPAD), lambda i: (i, 0)),
        out_shape=jax.ShapeDtypeStruct((_VOCAB, _DPAD), jnp.float32),
        compiler_params=pltpu.CompilerParams(
            dimension_semantics=("arbitrary",),
        ),
    )(emt)


def _reduce_kernel(x4, w1r, w2t):
    def body(x_ref, w1_ref, w2_ref, h_ref, s_ref, hs_ref):
        t = pl.program_id(1)

        @pl.when(t == 0)
        def _():
            x4v = x_ref[...]
            acc = jnp.zeros((_BB, _EMB), jnp.float32)
            for j in range(_CTX):
                xj = x4v[:, j].reshape(_BB, _DPAD).astype(jnp.bfloat16)
                acc = acc + jnp.dot(xj, w1_ref[j],
                                    preferred_element_type=jnp.float32)
            hb = jnp.transpose(jnp.maximum(acc, 0.0) * (1.0 / 16.0))
            hb = hb.astype(jnp.float8_e4m3fn)
            hs_ref[...] = hb
            h_ref[...] = hb
            s_ref[...] = jnp.full_like(s_ref, float(_VOCAB - _NT * _TV))

        logits = lax.dot_general(w2_ref[...], hs_ref[...],
                                 (((0,), (0,)), ((), ())),
                                 preferred_element_type=jnp.float32)
        e = jnp.exp(logits.astype(jnp.bfloat16)).astype(jnp.float32)
        e = jnp.sum(e, axis=0, keepdims=True)
        s_ref[...] = s_ref[...] + jnp.broadcast_to(e, (8, _BB))

    return pl.pallas_call(
        body,
        grid=(_NB, _NT),
        in_specs=[
            pl.BlockSpec((_B8, _CTX, 8, _DPAD), lambda b, t: (b, 0, 0, 0)),
            pl.BlockSpec((_CTX, _DPAD, _EMB), lambda b, t: (0, 0, 0)),
            pl.BlockSpec((_EMB, _TV), lambda b, t: (0, t)),
        ],
        out_specs=[
            pl.BlockSpec((_EMB, _BB), lambda b, t: (0, b)),
            pl.BlockSpec((8, _BB), lambda b, t: (0, b)),
        ],
        out_shape=[
            jax.ShapeDtypeStruct((_EMB, _BATCH), jnp.float8_e4m3fn),
            jax.ShapeDtypeStruct((8, _BATCH), jnp.float32),
        ],
        scratch_shapes=[pltpu.VMEM((_EMB, _BB), jnp.float8_e4m3fn)],
        compiler_params=pltpu.CompilerParams(
            dimension_semantics=("parallel", "arbitrary"),
        ),
    )(x4, w1r, w2t)


def _write_kernel(ht, s, w2t):
    def body(h_ref, s_ref, w2_ref, o_ref):
        logits = lax.dot_general(w2_ref[...], h_ref[...],
                                 (((0,), (0,)), ((), ())),
                                 preferred_element_type=jnp.float32)
        o_ref[...] = logits - jnp.log(s_ref[0:1, :])

    return pl.pallas_call(
        body,
        grid=(_NB, _NT),
        in_specs=[
            pl.BlockSpec((_EMB, _BB), lambda b, t: (0, b)),
            pl.BlockSpec((8, _BB), lambda b, t: (0, b)),
            pl.BlockSpec((_EMB, _TV), lambda b, t: (0, t)),
        ],
        out_specs=pl.BlockSpec((_TV, _BB), lambda b, t: (t, b)),
        out_shape=jax.ShapeDtypeStruct((_VOCAB, _BATCH), jnp.float32),
        compiler_params=pltpu.CompilerParams(
            dimension_semantics=("parallel", "arbitrary"),
        ),
    )(ht, s, w2t)


def kernel(inputs, em, W1, W2):
    idxp = jnp.transpose(inputs.reshape(_BATCH // 8, 8, _CTX), (0, 2, 1))
    idx3 = idxp.reshape(_NW, _NCHUNK, _CHUNK)
    em_pad = jnp.pad(em, ((0, 0), (0, _DPAD - _EMB)))
    rows = _sc_gather(em_pad, idx3)
    x4 = rows.reshape(_BATCH // 8, _CTX, 8, _DPAD)
    w1r = jnp.pad(W1.T.reshape(_CTX, _EMB, _EMB),
                  ((0, 0), (0, _DPAD - _EMB), (0, 0))).astype(jnp.bfloat16)
    w2t = (W2.T * 16.0).astype(jnp.float8_e4m3fn)
    w2t = jnp.pad(w2t, ((0, 0), (0, _NT * _TV - _VOCAB)))
    ht, s = _reduce_kernel(x4, w1r, w2t)
    outt = _write_kernel(ht, s, w2t)
    return outt.T


# speedup vs baseline: 2.8157x; 1.0023x over previous
import functools

import jax
import jax.numpy as jnp
from jax import lax
from jax.experimental import pallas as pl
from jax.experimental.pallas import tpu as pltpu
from jax.experimental.pallas import tpu_sc as plsc

_VOCAB = 100000
_EMB = 16
_CTX = 20
_BATCH = 1024
_NIDX = _CTX * _BATCH

_NC = 2
_NS = 16
_NW = _NC * _NS
_B_PER_W = _NIDX // _NW
_CHUNK = 128
_NCHUNK = _B_PER_W // _CHUNK
_DPAD = 128

_TV = 8192
_NT = (_VOCAB + _TV - 1) // _TV
_BB = 512
_NB = _BATCH // _BB
_B8 = _BB // 8


def _sc_gather(em, idx3):
    mesh = plsc.VectorSubcoreMesh(core_axis_name="c", subcore_axis_name="s")

    @functools.partial(
        pl.kernel,
        mesh=mesh,
        out_type=jax.ShapeDtypeStruct((_NIDX, _DPAD), jnp.float32),
        scratch_types=[
            pltpu.VMEM((_NCHUNK, _CHUNK), jnp.int32),
            pltpu.VMEM((_CHUNK, _DPAD), jnp.float32),
            pltpu.SemaphoreType.DMA,
        ],
    )
    def k(table_hbm, idx_hbm, out_hbm, idx_v, rows_v, sem):
        wid = lax.axis_index("s") * _NC + lax.axis_index("c")
        base = wid * _B_PER_W
        pltpu.sync_copy(idx_hbm.at[wid], idx_v)

        @pl.loop(0, _NCHUNK)
        def _(j):
            pltpu.async_copy(table_hbm.at[idx_v.at[j]], rows_v, sem).wait()
            pltpu.sync_copy(rows_v, out_hbm.at[pl.ds(base + j * _CHUNK, _CHUNK)])

    return k(em, idx3)


_TVC = 4096
_NTC = (_VOCAB + _TVC - 1) // _TVC


def _pad_table(emt):
    def body(e_ref, o_ref):
        o_ref[:, :_EMB] = jnp.transpose(e_ref[...])
        o_ref[:, _EMB:] = jnp.zeros((_TVC, _DPAD - _EMB), jnp.float32)

    return pl.pallas_call(
        body,
        grid=(_NTC,),
        in_specs=[pl.BlockSpec((_EMB, _TVC), lambda i: (0, i))],
        out_specs=pl.BlockSpec((_TVC, _DPAD), lambda i: (i, 0)),
        out_shape=jax.ShapeDtypeStruct((_VOCAB, _DPAD), jnp.float32),
        compiler_params=pltpu.CompilerParams(
            dimension_semantics=("arbitrary",),
        ),
    )(emt)


def _reduce_kernel(x4, w1r, w2t):
    def body(x_ref, w1_ref, w2_ref, h_ref, s_ref, hs_ref):
        t = pl.program_id(1)

        @pl.when(t == 0)
        def _():
            x4v = x_ref[...]
            acc = jnp.zeros((_BB, _EMB), jnp.float32)
            for j in range(_CTX):
                xj = x4v[:, j].reshape(_BB, _DPAD).astype(jnp.bfloat16)
                acc = acc + jnp.dot(xj, w1_ref[j],
                                    preferred_element_type=jnp.float32)
            hb = jnp.transpose(jnp.maximum(acc, 0.0) * (1.0 / 16.0))
            hb = hb.astype(jnp.float8_e4m3fn)
            hs_ref[...] = hb
            h_ref[...] = hb
            s_ref[...] = jnp.full_like(s_ref, float(_VOCAB - _NT * _TV))

        logits = lax.dot_general(w2_ref[...], hs_ref[...],
                                 (((0,), (0,)), ((), ())),
                                 preferred_element_type=jnp.float32)
        e = jnp.exp(logits.astype(jnp.bfloat16)).astype(jnp.float32)
        e = jnp.sum(e, axis=0, keepdims=True)
        s_ref[...] = s_ref[...] + jnp.broadcast_to(e, (8, _BB))

    return pl.pallas_call(
        body,
        grid=(_NB, _NT),
        in_specs=[
            pl.BlockSpec((_B8, _CTX, 8, _DPAD), lambda b, t: (b, 0, 0, 0)),
            pl.BlockSpec((_CTX, _DPAD, _EMB), lambda b, t: (0, 0, 0)),
            pl.BlockSpec((_EMB, _TV), lambda b, t: (0, t)),
        ],
        out_specs=[
            pl.BlockSpec((_EMB, _BB), lambda b, t: (0, b)),
            pl.BlockSpec((8, _BB), lambda b, t: (0, b)),
        ],
        out_shape=[
            jax.ShapeDtypeStruct((_EMB, _BATCH), jnp.float8_e4m3fn),
            jax.ShapeDtypeStruct((8, _BATCH), jnp.float32),
        ],
        scratch_shapes=[pltpu.VMEM((_EMB, _BB), jnp.float8_e4m3fn)],
        compiler_params=pltpu.CompilerParams(
            dimension_semantics=("parallel", "arbitrary"),
        ),
    )(x4, w1r, w2t)


def _write_kernel(ht, s, w2t):
    def body(h_ref, s_ref, w2_ref, o_ref):
        logits = lax.dot_general(w2_ref[...], h_ref[...],
                                 (((0,), (0,)), ((), ())),
                                 preferred_element_type=jnp.float32)
        o_ref[...] = logits - jnp.log(s_ref[0:1, :])

    return pl.pallas_call(
        body,
        grid=(_NB, _NT),
        in_specs=[
            pl.BlockSpec((_EMB, _BB), lambda b, t: (0, b)),
            pl.BlockSpec((8, _BB), lambda b, t: (0, b)),
            pl.BlockSpec((_EMB, _TV), lambda b, t: (0, t)),
        ],
        out_specs=pl.BlockSpec((_TV, _BB), lambda b, t: (t, b)),
        out_shape=jax.ShapeDtypeStruct((_VOCAB, _BATCH), jnp.float32),
        compiler_params=pltpu.CompilerParams(
            dimension_semantics=("parallel", "arbitrary"),
        ),
    )(ht, s, w2t)


def kernel(inputs, em, W1, W2):
    idxp = jnp.transpose(inputs.reshape(_BATCH // 8, 8, _CTX), (0, 2, 1))
    idx3 = idxp.reshape(_NW, _NCHUNK, _CHUNK)
    em_pad = jnp.pad(em, ((0, 0), (0, _DPAD - _EMB)))
    rows = _sc_gather(em_pad, idx3)
    x4 = rows.reshape(_BATCH // 8, _CTX, 8, _DPAD)
    w1r = jnp.pad(W1.T.reshape(_CTX, _EMB, _EMB),
                  ((0, 0), (0, _DPAD - _EMB), (0, 0))).astype(jnp.bfloat16)
    w2t = (W2.T * 16.0).astype(jnp.float8_e4m3fn)
    w2t = jnp.pad(w2t, ((0, 0), (0, _NT * _TV - _VOCAB)))
    ht, s = _reduce_kernel(x4, w1r, w2t)
    outt = _write_kernel(ht, s, w2t)
    return outt.T
